# Initial kernel scaffold; baseline (speedup 1.0000x reference)
import jax
import jax.numpy as jnp
from jax.experimental import pallas as pl


def kernel(edge_index, features, batch, i, w1a, b1a, w1b, b1b, bn1_g, bn1_b, eps1, w2a, b2a, w2b, b2b, bn2_g, bn2_b, eps2, w3a, b3a, w3b, b3b, bn3_g, bn3_b, eps3, a3w1, a3b1, a3w2, a3b2, a2w1, a2b1, a2w2, a2b2, a1w1, a1b1, a1w2, a1b2):
    raise NotImplementedError("write your pallas kernel here")


# speedup vs baseline: 3.9472x; 3.9472x over previous
import functools

import jax
import jax.numpy as jnp
from jax import lax
from jax.experimental import pallas as pl
from jax.experimental.pallas import tpu as pltpu
from jax.experimental.pallas import tpu_sc as plsc

NN = 10000
NE = 160000
NG = 128
RB = 1000
NBLK = NN // RB
EB = 128
NEB = NE // EB
NSUB = 16
RPS = NN // NSUB
BPC = NEB // 2
HI = lax.Precision.HIGHEST


def _dot(a, b):
    return lax.dot_general(a, b, (((1,), (0,)), ((), ())),
                           precision=HI, preferred_element_type=jnp.float32)


def _dot_t(a, b):
    return lax.dot_general(a, b, (((0,), (0,)), ((), ())),
                           precision=HI, preferred_element_type=jnp.float32)


def _dot_rt(a, b):
    return lax.dot_general(a, b, (((1,), (1,)), ((), ())),
                           precision=HI, preferred_element_type=jnp.float32)


NNP = 10240
RPSP = NNP // NSUB


@functools.cache
def _sc_agg(n_chunks):
    mesh = plsc.VectorSubcoreMesh(core_axis_name="c", subcore_axis_name="s")

    def body(src_hbm, dst_hbm, *rest):
        zs = rest[:n_chunks]
        out_hbm = rest[n_chunks]
        srcb, dstb, rows, zbuf, agg, sem = rest[n_chunks + 1:]
        cid = lax.axis_index("c")
        sid = lax.axis_index("s")

        zero16 = jnp.zeros((16,), jnp.float32)

        @pl.loop(0, 128)
        def _(r):
            @pl.loop(0, 128, step=16)
            def _(c0):
                zbuf[r, pl.ds(c0, 16)] = zero16

        for c in range(n_chunks):
            for j in range(5):
                pltpu.sync_copy(zbuf, agg.at[pl.ds(sid * RPSP + j * 128, 128)])
            plsc.subcore_barrier()

            @pl.loop(cid * BPC + sid, (cid + 1) * BPC, step=NSUB)
            def _(b):
                pltpu.sync_copy(src_hbm.at[pl.ds(b * EB, EB)], srcb)
                pltpu.sync_copy(dst_hbm.at[pl.ds(b * EB, EB)], dstb)
                pltpu.async_copy(zs[c].at[srcb], rows, sem).wait()
                pltpu.sync_copy(rows, agg.at[dstb], add=True)

            plsc.subcore_barrier()

            @pl.when(sid < NSUB - 1)
            def _():
                pltpu.sync_copy(agg.at[pl.ds(sid * RPSP, RPSP)],
                                out_hbm.at[cid, c, pl.ds(sid * RPSP, RPSP)])

            @pl.when(sid == NSUB - 1)
            def _():
                pltpu.sync_copy(agg.at[pl.ds(9600, 400)],
                                out_hbm.at[cid, c, pl.ds(9600, 400)])

    return pl.kernel(
        body,
        out_type=jax.ShapeDtypeStruct((2, n_chunks, NN, 128), jnp.float32),
        mesh=mesh,
        scratch_types=[
            pltpu.VMEM((EB,), jnp.int32),
            pltpu.VMEM((EB,), jnp.int32),
            pltpu.VMEM((EB, 128), jnp.float32),
            pltpu.VMEM((128, 128), jnp.float32),
            pltpu.VMEM_SHARED((NNP, 128), jnp.float32),
            pltpu.SemaphoreType.DMA,
        ],
    )


def _gin12(n_cin, f_out, with_z):
    fo_chunks = f_out // 128

    def body(*refs):
        xs = refs[:n_cin]
        k = n_cin
        agg_ref, eps_ref, wa_ref, ba_ref, wb_ref, bb_ref, g_ref, b2_ref = \
            refs[k:k + 8]
        k += 8
        wz_ref = None
        if with_z:
            wz_ref = refs[k]
            k += 1
        outs = refs[k:k + fo_chunks]
        k += fo_chunks
        zouts = refs[k:k + 2] if with_z else ()
        k += 2 if with_z else 0
        h_scr, st_scr = refs[k:k + 2]

        p = pl.program_id(0)
        i = pl.program_id(1)

        @pl.when(p == 0)
        def _():
            eps = eps_ref[0, 0]
            agg = agg_ref[...]
            wa = wa_ref[...]
            t = None
            for c in range(n_cin):
                u = (1.0 + eps) * xs[c][...] + agg[0, c] + agg[1, c]
                tc = _dot(u, wa[c * 128:(c + 1) * 128, :])
                t = tc if t is None else t + tc
            t = jnp.maximum(t + ba_ref[...], 0.0)
            h = _dot(t, wb_ref[...]) + bb_ref[...]
            h_scr[pl.ds(i * RB, RB), :] = h
            s = jnp.sum(h, axis=0, keepdims=True)
            sq = jnp.sum(h * h, axis=0, keepdims=True)

            @pl.when(i == 0)
            def _():
                st_scr[0:1, :] = s
                st_scr[1:2, :] = sq

            @pl.when(i > 0)
            def _():
                st_scr[0:1, :] += s
                st_scr[1:2, :] += sq

        @pl.when(p == 1)
        def _():
            mu = st_scr[0:1, :] * (1.0 / NN)
            var = st_scr[1:2, :] * (1.0 / NN) - mu * mu
            rs = lax.rsqrt(var + 1e-5)
            h = h_scr[pl.ds(i * RB, RB), :]
            f = jnp.maximum(g_ref[...] * (h - mu) * rs + b2_ref[...], 0.0)
            for c in range(fo_chunks):
                outs[c][...] = f[:, c * 128:(c + 1) * 128]
            if with_z:
                z = _dot(f, wz_ref[...])
                for c in range(2):
                    zouts[c][...] = z[:, c * 128:(c + 1) * 128]

    in_specs = (
        [pl.BlockSpec((RB, 128), lambda p, i: (i, 0)) for _ in range(n_cin)]
        + [pl.BlockSpec((2, n_cin, RB, 128), lambda p, i: (0, 0, i, 0)),
           pl.BlockSpec(memory_space=pltpu.SMEM),
           pl.BlockSpec((n_cin * 128, f_out), lambda p, i: (0, 0)),
           pl.BlockSpec((1, f_out), lambda p, i: (0, 0)),
           pl.BlockSpec((f_out, f_out), lambda p, i: (0, 0)),
           pl.BlockSpec((1, f_out), lambda p, i: (0, 0)),
           pl.BlockSpec((1, f_out), lambda p, i: (0, 0)),
           pl.BlockSpec((1, f_out), lambda p, i: (0, 0))]
    )
    out_specs = [pl.BlockSpec((RB, 128), lambda p, i: (i, 0))
                 for _ in range(fo_chunks)]
    out_shape = [jax.ShapeDtypeStruct((NN, 128), jnp.float32)
                 for _ in range(fo_chunks)]
    if with_z:
        in_specs.append(pl.BlockSpec((f_out, 256), lambda p, i: (0, 0)))
        out_specs += [pl.BlockSpec((RB, 128), lambda p, i: (i, 0))
                      for _ in range(2)]
        out_shape += [jax.ShapeDtypeStruct((NN, 128), jnp.float32)
                      for _ in range(2)]

    return pl.pallas_call(
        body,
        grid=(2, NBLK),
        in_specs=in_specs,
        out_specs=out_specs,
        out_shape=out_shape,
        scratch_shapes=[pltpu.VMEM((NN, f_out), jnp.float32),
                        pltpu.VMEM((2, f_out), jnp.float32)],
    )


def _gin3():
    def body(z0, z1, agg_ref, eps_ref, ba_ref, wb_ref, bb_ref, g_ref, b2_ref,
             out_ref, h_scr, st_scr):
        zs = (z0, z1)
        p = pl.program_id(0)
        i = pl.program_id(1)

        @pl.when(p == 0)
        def _():
            eps = eps_ref[0, 0]
            agg = agg_ref[...]
            ba = ba_ref[...]
            wb = wb_ref[...]
            t = None
            for c in range(2):
                pre = ((1.0 + eps) * zs[c][...] + agg[0, c] + agg[1, c]
                       + ba[:, c * 128:(c + 1) * 128])
                tc = _dot(jnp.maximum(pre, 0.0), wb[c * 128:(c + 1) * 128, :])
                t = tc if t is None else t + tc
            h = t + bb_ref[...]
            h_scr[pl.ds(i * RB, RB), :] = h
            s = jnp.sum(h, axis=0, keepdims=True)
            sq = jnp.sum(h * h, axis=0, keepdims=True)

            @pl.when(i == 0)
            def _():
                st_scr[0:1, :] = s
                st_scr[1:2, :] = sq

            @pl.when(i > 0)
            def _():
                st_scr[0:1, :] += s
                st_scr[1:2, :] += sq

        @pl.when(p == 1)
        def _():
            mu = st_scr[0:1, :] * (1.0 / NN)
            var = st_scr[1:2, :] * (1.0 / NN) - mu * mu
            rs = lax.rsqrt(var + 1e-5)
            h = h_scr[pl.ds(i * RB, RB), :]
            out_ref[...] = g_ref[...] * (h - mu) * rs + b2_ref[...]

    return pl.pallas_call(
        body,
        grid=(2, NBLK),
        in_specs=[pl.BlockSpec((RB, 128), lambda p, i: (i, 0)),
                  pl.BlockSpec((RB, 128), lambda p, i: (i, 0)),
                  pl.BlockSpec((2, 2, RB, 128), lambda p, i: (0, 0, i, 0)),
                  pl.BlockSpec(memory_space=pltpu.SMEM),
                  pl.BlockSpec((1, 256), lambda p, i: (0, 0)),
                  pl.BlockSpec((256, 256), lambda p, i: (0, 0)),
                  pl.BlockSpec((1, 256), lambda p, i: (0, 0)),
                  pl.BlockSpec((1, 256), lambda p, i: (0, 0)),
                  pl.BlockSpec((1, 256), lambda p, i: (0, 0))],
        out_specs=pl.BlockSpec((RB, 256), lambda p, i: (i, 0)),
        out_shape=jax.ShapeDtypeStruct((NN, 256), jnp.float32),
        scratch_shapes=[pltpu.VMEM((NN, 256), jnp.float32),
                        pltpu.VMEM((2, 256), jnp.float32)],
    )


def _att(f_dim, n_ch):
    f4 = f_dim // 4

    def body(*refs):
        fs = refs[:n_ch]
        batch_ref, w1_ref, b1_ref, w2_ref, b2_ref = refs[n_ch:n_ch + 5]
        out_ref = refs[n_ch + 5]
        sums, cnt, tg, pacc = refs[n_ch + 6:n_ch + 10]

        p = pl.program_id(0)
        i = pl.program_id(1)

        if n_ch > 1:
            f = jnp.concatenate([r[...] for r in fs], axis=1)
        else:
            f = fs[0][...]
        bvec = batch_ref[...]
        cols = lax.broadcasted_iota(jnp.int32, (RB, NG), 1)
        onehot = (bvec == cols).astype(jnp.float32)

        @pl.when(p == 0)
        def _():
            a = jnp.maximum(_dot(f, w1_ref[...]) + b1_ref[...], 0.0)
            att = jnp.tanh(_dot(a, w2_ref[...]) + b2_ref[...])
            sblk = _dot_t(onehot, att * f)
            cblk = _dot_t(onehot, jnp.ones((RB, 1), jnp.float32))

            @pl.when(i == 0)
            def _():
                sums[...] = sblk
                cnt[...] = cblk

            @pl.when(i > 0)
            def _():
                sums[...] += sblk
                cnt[...] += cblk

        @pl.when(p == 1)
        def _():
            @pl.when(i == 0)
            def _():
                mean = sums[...] / jnp.maximum(cnt[...], 1.0)
                tg[...] = jnp.tanh(mean)

            s = _dot_rt(f, tg[...])
            s = jnp.sum(s * onehot, axis=1, keepdims=True)
            coef = jax.nn.sigmoid(10.0 * s)
            pblk = _dot_t(onehot, coef * f)

            @pl.when(i == 0)
            def _():
                pacc[...] = pblk

            @pl.when(i > 0)
            def _():
                pacc[...] += pblk

            @pl.when(i == NBLK - 1)
            def _():
                out_ref[...] = pacc[...]

    return pl.pallas_call(
        body,
        grid=(2, NBLK),
        in_specs=(
            [pl.BlockSpec((RB, 128 if n_ch > 1 else f_dim),
                          lambda p, i: (i, 0)) for _ in range(n_ch)]
            + [pl.BlockSpec((RB, 1), lambda p, i: (i, 0)),
               pl.BlockSpec((f_dim, f4), lambda p, i: (0, 0)),
               pl.BlockSpec((1, f4), lambda p, i: (0, 0)),
               pl.BlockSpec((f4, f_dim), lambda p, i: (0, 0)),
               pl.BlockSpec((1, f_dim), lambda p, i: (0, 0))]
        ),
        out_specs=pl.BlockSpec((NG, f_dim), lambda p, i: (0, 0)),
        out_shape=jax.ShapeDtypeStruct((NG, f_dim), jnp.float32),
        scratch_shapes=[pltpu.VMEM((NG, f_dim), jnp.float32),
                        pltpu.VMEM((NG, 1), jnp.float32),
                        pltpu.VMEM((NG, f_dim), jnp.float32),
                        pltpu.VMEM((NG, f_dim), jnp.float32)],
    )


def kernel(edge_index, features, batch, i,
           w1a, b1a, w1b, b1b, bn1_g, bn1_b, eps1,
           w2a, b2a, w2b, b2b, bn2_g, bn2_b, eps2,
           w3a, b3a, w3b, b3b, bn3_g, bn3_b, eps3,
           a3w1, a3b1, a3w2, a3b2,
           a2w1, a2b1, a2w2, a2b2,
           a1w1, a1b1, a1w2, a1b2):
    del i
    src2 = edge_index[0]
    dst2 = edge_index[1]
    batch2 = batch.reshape(NN, 1)
    row = lambda v: v.reshape(1, -1)
    sca = lambda v: v.reshape(1, 1)

    agg1 = _sc_agg(1)(src2, dst2, features)
    f1c = _gin12(1, 512, False)(
        features, agg1, sca(eps1), w1a, row(b1a), w1b, row(b1b),
        row(bn1_g), row(bn1_b))

    agg2 = _sc_agg(4)(src2, dst2, *f1c)
    out2 = _gin12(4, 512, True)(
        *f1c, agg2, sca(eps2), w2a, row(b2a), w2b, row(b2b),
        row(bn2_g), row(bn2_b), w3a)
    f2c, z3c = out2[:4], out2[4:]

    agg3 = _sc_agg(2)(src2, dst2, *z3c)
    f3 = _gin3()(
        *z3c, agg3, sca(eps3), row(b3a), w3b, row(b3b),
        row(bn3_g), row(bn3_b))

    p3 = _att(256, 1)(f3, batch2, a3w1, row(a3b1), a3w2, row(a3b2))
    p2 = _att(512, 4)(*f2c, batch2, a2w1, row(a2b1), a2w2, row(a2b2))
    p1 = _att(512, 4)(*f1c, batch2, a1w1, row(a1b1), a1w2, row(a1b2))
    return jnp.concatenate((p3, p2, p1), axis=1)


# speedup vs baseline: 5.5623x; 1.4092x over previous
import functools

import jax
import jax.numpy as jnp
from jax import lax
from jax.experimental import pallas as pl
from jax.experimental.pallas import tpu as pltpu
from jax.experimental.pallas import tpu_sc as plsc

NN = 10000
NE = 160000
NG = 128
RB = 1000
NBLK = NN // RB
EB = 128
NEB = NE // EB
NEBP = 1280
NSUB = 16
RPS = NN // NSUB
BPC = NEB // 2
HI = None


def _dot(a, b):
    return lax.dot_general(a, b, (((1,), (0,)), ((), ())),
                           precision=HI, preferred_element_type=jnp.float32)


def _dot_t(a, b):
    return lax.dot_general(a, b, (((0,), (0,)), ((), ())),
                           precision=HI, preferred_element_type=jnp.float32)


def _dot_rt(a, b):
    return lax.dot_general(a, b, (((1,), (1,)), ((), ())),
                           precision=HI, preferred_element_type=jnp.float32)


NNP = 10240
RPSP = NNP // NSUB


@functools.cache
def _sc_agg(n_chunks):
    mesh = plsc.VectorSubcoreMesh(core_axis_name="c", subcore_axis_name="s")

    def body(src_hbm, dst_hbm, *rest):
        zs = rest[:n_chunks]
        out_hbm = rest[n_chunks]
        (sbuf, dbuf, rows0, rows1, zbuf, agg,
         isem, zsem, gsem0, gsem1, ssem) = rest[n_chunks + 1:]
        cid = lax.axis_index("c")
        sid = lax.axis_index("s")
        rows = (rows0, rows1)
        gsem = (gsem0, gsem1)

        wid = cid * NSUB + sid
        last = wid == 2 * NSUB - 1
        lastrow = sid == NSUB - 1
        gbase = pl.multiple_of(jnp.where(last, 1240, 40 * wid), 8)
        nblk = jnp.where(last, 10, 40)
        npair = jnp.where(last, 5, 20)

        @pl.when(jnp.logical_not(last))
        def _():
            pltpu.async_copy(src_hbm.at[pl.ds(gbase, 40)], sbuf, isem).wait()
            pltpu.async_copy(dst_hbm.at[pl.ds(gbase, 40)], dbuf, isem).wait()

        @pl.when(last)
        def _():
            pltpu.async_copy(src_hbm.at[pl.ds(gbase, 16)],
                             sbuf.at[pl.ds(0, 16)], isem).wait()
            pltpu.async_copy(dst_hbm.at[pl.ds(gbase, 16)],
                             dbuf.at[pl.ds(0, 16)], isem).wait()

        zero16 = jnp.zeros((16,), jnp.float32)

        @pl.loop(0, 32)
        def _(r):
            @pl.loop(0, 128, step=16)
            def _(c0):
                zbuf[r, pl.ds(c0, 16)] = zero16

        for c in range(n_chunks):
            for j in range(20):
                pltpu.async_copy(
                    zbuf, agg.at[pl.ds(sid * RPSP + j * 32, 32)], zsem)
            for j in range(20):
                pltpu.make_async_copy(
                    zbuf, agg.at[pl.ds(sid * RPSP, 32)], zsem).wait()
            plsc.subcore_barrier()

            @pl.loop(0, nblk)
            def _(k):
                pltpu.async_copy(zs[c].at[sbuf.at[k]], rows0, gsem0).wait()
                pltpu.sync_copy(rows0, agg.at[dbuf.at[k]], add=True)

            plsc.subcore_barrier()

            @pl.when(jnp.logical_not(lastrow))
            def _():
                pltpu.sync_copy(agg.at[pl.ds(sid * RPSP, RPSP)],
                                out_hbm.at[cid, c, pl.ds(sid * RPSP, RPSP)])

            @pl.when(lastrow)
            def _():
                pltpu.sync_copy(agg.at[pl.ds(9600, 400)],
                                out_hbm.at[cid, c, pl.ds(9600, 400)])

    return pl.kernel(
        body,
        out_type=jax.ShapeDtypeStruct((2, n_chunks, NN, 128), jnp.float32),
        mesh=mesh,
        scratch_types=[
            pltpu.VMEM((40, EB), jnp.int32),
            pltpu.VMEM((40, EB), jnp.int32),
            pltpu.VMEM((EB, 128), jnp.float32),
            pltpu.VMEM((EB, 128), jnp.float32),
            pltpu.VMEM((32, 128), jnp.float32),
            pltpu.VMEM_SHARED((NNP, 128), jnp.float32),
            pltpu.SemaphoreType.DMA,
            pltpu.SemaphoreType.DMA,
            pltpu.SemaphoreType.DMA,
            pltpu.SemaphoreType.DMA,
            pltpu.SemaphoreType.DMA,
        ],
    )


def _gin12(n_cin, f_out, with_z):
    fo_chunks = f_out // 128

    def body(*refs):
        xs = refs[:n_cin]
        k = n_cin
        agg_ref, eps_ref, wa_ref, ba_ref, wb_ref, bb_ref, g_ref, b2_ref = \
            refs[k:k + 8]
        k += 8
        wz_ref = None
        if with_z:
            wz_ref = refs[k]
            k += 1
        outs = refs[k:k + fo_chunks]
        k += fo_chunks
        zouts = refs[k:k + 2] if with_z else ()
        k += 2 if with_z else 0
        h_scr, st_scr = refs[k:k + 2]

        p = pl.program_id(0)
        i = pl.program_id(1)

        @pl.when(p == 0)
        def _():
            eps = eps_ref[0, 0]
            agg = agg_ref[...]
            wa = wa_ref[...]
            t = None
            for c in range(n_cin):
                u = (1.0 + eps) * xs[c][...] + agg[0, c] + agg[1, c]
                tc = _dot(u, wa[c * 128:(c + 1) * 128, :])
                t = tc if t is None else t + tc
            t = jnp.maximum(t + ba_ref[...], 0.0)
            h = _dot(t, wb_ref[...]) + bb_ref[...]
            h_scr[pl.ds(i * RB, RB), :] = h
            s = jnp.sum(h, axis=0, keepdims=True)

            @pl.when(i == 0)
            def _():
                st_scr[0:1, :] = s

            @pl.when(i > 0)
            def _():
                st_scr[0:1, :] += s

        @pl.when(p == 1)
        def _():
            mu = st_scr[0:1, :] * (1.0 / NN)
            d = h_scr[pl.ds(i * RB, RB), :] - mu
            sq = jnp.sum(d * d, axis=0, keepdims=True)

            @pl.when(i == 0)
            def _():
                st_scr[1:2, :] = sq

            @pl.when(i > 0)
            def _():
                st_scr[1:2, :] += sq

        @pl.when(p == 2)
        def _():
            mu = st_scr[0:1, :] * (1.0 / NN)
            var = st_scr[1:2, :] * (1.0 / NN)
            rs = lax.rsqrt(var + 1e-5)
            h = h_scr[pl.ds(i * RB, RB), :]
            f = jnp.maximum(g_ref[...] * (h - mu) * rs + b2_ref[...], 0.0)
            for c in range(fo_chunks):
                outs[c][...] = f[:, c * 128:(c + 1) * 128]
            if with_z:
                z = _dot(f, wz_ref[...])
                for c in range(2):
                    zouts[c][...] = z[:, c * 128:(c + 1) * 128]

    in_specs = (
        [pl.BlockSpec((RB, 128), lambda p, i: (i, 0)) for _ in range(n_cin)]
        + [pl.BlockSpec((2, n_cin, RB, 128), lambda p, i: (0, 0, i, 0)),
           pl.BlockSpec(memory_space=pltpu.SMEM),
           pl.BlockSpec((n_cin * 128, f_out), lambda p, i: (0, 0)),
           pl.BlockSpec((1, f_out), lambda p, i: (0, 0)),
           pl.BlockSpec((f_out, f_out), lambda p, i: (0, 0)),
           pl.BlockSpec((1, f_out), lambda p, i: (0, 0)),
           pl.BlockSpec((1, f_out), lambda p, i: (0, 0)),
           pl.BlockSpec((1, f_out), lambda p, i: (0, 0))]
    )
    out_specs = [pl.BlockSpec((RB, 128), lambda p, i: (i, 0))
                 for _ in range(fo_chunks)]
    out_shape = [jax.ShapeDtypeStruct((NN, 128), jnp.float32)
                 for _ in range(fo_chunks)]
    if with_z:
        in_specs.append(pl.BlockSpec((f_out, 256), lambda p, i: (0, 0)))
        out_specs += [pl.BlockSpec((RB, 128), lambda p, i: (i, 0))
                      for _ in range(2)]
        out_shape += [jax.ShapeDtypeStruct((NN, 128), jnp.float32)
                      for _ in range(2)]

    return pl.pallas_call(
        body,
        grid=(3, NBLK),
        in_specs=in_specs,
        out_specs=out_specs,
        out_shape=out_shape,
        scratch_shapes=[pltpu.VMEM((NN, f_out), jnp.float32),
                        pltpu.VMEM((2, f_out), jnp.float32)],
    )


def _gin3():
    def body(z0, z1, agg_ref, eps_ref, ba_ref, wb_ref, bb_ref, g_ref, b2_ref,
             out_ref, h_scr, st_scr):
        zs = (z0, z1)
        p = pl.program_id(0)
        i = pl.program_id(1)

        @pl.when(p == 0)
        def _():
            eps = eps_ref[0, 0]
            agg = agg_ref[...]
            ba = ba_ref[...]
            wb = wb_ref[...]
            t = None
            for c in range(2):
                pre = ((1.0 + eps) * zs[c][...] + agg[0, c] + agg[1, c]
                       + ba[:, c * 128:(c + 1) * 128])
                tc = _dot(jnp.maximum(pre, 0.0), wb[c * 128:(c + 1) * 128, :])
                t = tc if t is None else t + tc
            h = t + bb_ref[...]
            h_scr[pl.ds(i * RB, RB), :] = h
            s = jnp.sum(h, axis=0, keepdims=True)

            @pl.when(i == 0)
            def _():
                st_scr[0:1, :] = s

            @pl.when(i > 0)
            def _():
                st_scr[0:1, :] += s

        @pl.when(p == 1)
        def _():
            mu = st_scr[0:1, :] * (1.0 / NN)
            d = h_scr[pl.ds(i * RB, RB), :] - mu
            sq = jnp.sum(d * d, axis=0, keepdims=True)

            @pl.when(i == 0)
            def _():
                st_scr[1:2, :] = sq

            @pl.when(i > 0)
            def _():
                st_scr[1:2, :] += sq

        @pl.when(p == 2)
        def _():
            mu = st_scr[0:1, :] * (1.0 / NN)
            var = st_scr[1:2, :] * (1.0 / NN)
            rs = lax.rsqrt(var + 1e-5)
            h = h_scr[pl.ds(i * RB, RB), :]
            out_ref[...] = g_ref[...] * (h - mu) * rs + b2_ref[...]

    return pl.pallas_call(
        body,
        grid=(3, NBLK),
        in_specs=[pl.BlockSpec((RB, 128), lambda p, i: (i, 0)),
                  pl.BlockSpec((RB, 128), lambda p, i: (i, 0)),
                  pl.BlockSpec((2, 2, RB, 128), lambda p, i: (0, 0, i, 0)),
                  pl.BlockSpec(memory_space=pltpu.SMEM),
                  pl.BlockSpec((1, 256), lambda p, i: (0, 0)),
                  pl.BlockSpec((256, 256), lambda p, i: (0, 0)),
                  pl.BlockSpec((1, 256), lambda p, i: (0, 0)),
                  pl.BlockSpec((1, 256), lambda p, i: (0, 0)),
                  pl.BlockSpec((1, 256), lambda p, i: (0, 0))],
        out_specs=pl.BlockSpec((RB, 256), lambda p, i: (i, 0)),
        out_shape=jax.ShapeDtypeStruct((NN, 256), jnp.float32),
        scratch_shapes=[pltpu.VMEM((NN, 256), jnp.float32),
                        pltpu.VMEM((2, 256), jnp.float32)],
    )


def _att(f_dim, n_ch):
    f4 = f_dim // 4

    def body(*refs):
        fs = refs[:n_ch]
        batch_ref, w1_ref, b1_ref, w2_ref, b2_ref = refs[n_ch:n_ch + 5]
        out_ref = refs[n_ch + 5]
        sums, cnt, tg, pacc = refs[n_ch + 6:n_ch + 10]

        p = pl.program_id(0)
        i = pl.program_id(1)

        if n_ch > 1:
            f = jnp.concatenate([r[...] for r in fs], axis=1)
        else:
            f = fs[0][...]
        bvec = batch_ref[...]
        cols = lax.broadcasted_iota(jnp.int32, (RB, NG), 1)
        onehot = (bvec == cols).astype(jnp.float32)

        @pl.when(p == 0)
        def _():
            a = jnp.maximum(_dot(f, w1_ref[...]) + b1_ref[...], 0.0)
            att = jnp.tanh(_dot(a, w2_ref[...]) + b2_ref[...])
            sblk = _dot_t(onehot, att * f)
            cblk = _dot_t(onehot, jnp.ones((RB, 1), jnp.float32))

            @pl.when(i == 0)
            def _():
                sums[...] = sblk
                cnt[...] = cblk

            @pl.when(i > 0)
            def _():
                sums[...] += sblk
                cnt[...] += cblk

        @pl.when(p == 1)
        def _():
            @pl.when(i == 0)
            def _():
                mean = sums[...] / jnp.maximum(cnt[...], 1.0)
                tg[...] = jnp.tanh(mean)

            s = _dot_rt(f, tg[...])
            s = jnp.sum(s * onehot, axis=1, keepdims=True)
            coef = jax.nn.sigmoid(10.0 * s)
            pblk = _dot_t(onehot, coef * f)

            @pl.when(i == 0)
            def _():
                pacc[...] = pblk

            @pl.when(i > 0)
            def _():
                pacc[...] += pblk

            @pl.when(i == NBLK - 1)
            def _():
                out_ref[...] = pacc[...]

    return pl.pallas_call(
        body,
        grid=(2, NBLK),
        in_specs=(
            [pl.BlockSpec((RB, 128 if n_ch > 1 else f_dim),
                          lambda p, i: (i, 0)) for _ in range(n_ch)]
            + [pl.BlockSpec((RB, 1), lambda p, i: (i, 0)),
               pl.BlockSpec((f_dim, f4), lambda p, i: (0, 0)),
               pl.BlockSpec((1, f4), lambda p, i: (0, 0)),
               pl.BlockSpec((f4, f_dim), lambda p, i: (0, 0)),
               pl.BlockSpec((1, f_dim), lambda p, i: (0, 0))]
        ),
        out_specs=pl.BlockSpec((NG, f_dim), lambda p, i: (0, 0)),
        out_shape=jax.ShapeDtypeStruct((NG, f_dim), jnp.float32),
        scratch_shapes=[pltpu.VMEM((NG, f_dim), jnp.float32),
                        pltpu.VMEM((NG, 1), jnp.float32),
                        pltpu.VMEM((NG, f_dim), jnp.float32),
                        pltpu.VMEM((NG, f_dim), jnp.float32)],
    )


def kernel(edge_index, features, batch, i,
           w1a, b1a, w1b, b1b, bn1_g, bn1_b, eps1,
           w2a, b2a, w2b, b2b, bn2_g, bn2_b, eps2,
           w3a, b3a, w3b, b3b, bn3_g, bn3_b, eps3,
           a3w1, a3b1, a3w2, a3b2,
           a2w1, a2b1, a2w2, a2b2,
           a1w1, a1b1, a1w2, a1b2):
    del i
    pad = ((0, NEBP - NEB), (0, 0))
    src2 = jnp.pad(edge_index[0].reshape(NEB, EB), pad)
    dst2 = jnp.pad(edge_index[1].reshape(NEB, EB), pad)
    batch2 = batch.reshape(NN, 1)
    row = lambda v: v.reshape(1, -1)
    sca = lambda v: v.reshape(1, 1)

    agg1 = _sc_agg(1)(src2, dst2, features)
    f1c = _gin12(1, 512, False)(
        features, agg1, sca(eps1), w1a, row(b1a), w1b, row(b1b),
        row(bn1_g), row(bn1_b))

    agg2 = _sc_agg(4)(src2, dst2, *f1c)
    out2 = _gin12(4, 512, True)(
        *f1c, agg2, sca(eps2), w2a, row(b2a), w2b, row(b2b),
        row(bn2_g), row(bn2_b), w3a)
    f2c, z3c = out2[:4], out2[4:]

    agg3 = _sc_agg(2)(src2, dst2, *z3c)
    f3 = _gin3()(
        *z3c, agg3, sca(eps3), row(b3a), w3b, row(b3b),
        row(bn3_g), row(bn3_b))

    p3 = _att(256, 1)(f3, batch2, a3w1, row(a3b1), a3w2, row(a3b2))
    p2 = _att(512, 4)(*f2c, batch2, a2w1, row(a2b1), a2w2, row(a2b2))
    p1 = _att(512, 4)(*f1c, batch2, a1w1, row(a1b1), a1w2, row(a1b2))
    return jnp.concatenate((p3, p2, p1), axis=1)
